# Initial kernel scaffold; baseline (speedup 1.0000x reference)
import jax
import jax.numpy as jnp
from jax import lax
from jax.experimental import pallas as pl
from jax.experimental.pallas import tpu as pltpu
from jax.experimental.pallas import tpu_sc as plsc

BATCH = 4096
HIST = 200
NDIM = 16
LANES = 16
NWORKERS = 32
ROWS_PER_W = BATCH // NWORKERS

_NCHUNKS = HIST // LANES + 1


def _mirt_body(abil_hbm, ids_hbm, disc_hbm, diff_hbm, out_hbm,
               idx_v, rows_v, diff_v, out_v, abil_v,
               sem_rows, sem_diff):
    nc = lax.axis_size("c")
    wid = lax.axis_index("s") * nc + lax.axis_index("c")
    base = wid * ROWS_PER_W

    pltpu.sync_copy(abil_hbm.at[pl.ds(base, ROWS_PER_W)], abil_v)

    def row_body(r, carry):
        row = base + r
        pltpu.sync_copy(ids_hbm.at[row], idx_v)
        cp_rows = pltpu.make_async_copy(disc_hbm.at[idx_v], rows_v, sem_rows)
        cp_diff = pltpu.make_async_copy(diff_hbm.at[idx_v], diff_v, sem_diff)
        cp_rows.start()
        cp_diff.start()
        cp_rows.wait()
        cp_diff.wait()

        abil = abil_v[r]

        def chunk_body(c, carry2):
            off = jnp.minimum(c * LANES, HIST - LANES)
            acc = jnp.zeros((LANES,), jnp.float32)
            lane = lax.iota(jnp.int32, LANES)
            for j in range(LANES):
                w = rows_v[off + j]
                s = jnp.sum(jnp.abs(w) * abil)
                acc = jnp.where(lane == j, s, acc)
            x = acc - diff_v[pl.ds(off, LANES)]
            out_v[pl.ds(off, LANES)] = 1.0 / (1.0 + jnp.exp(-x))
            return carry2

        lax.fori_loop(0, _NCHUNKS, chunk_body, 0)
        pltpu.sync_copy(out_v, out_hbm.at[row])
        return carry

    lax.fori_loop(0, ROWS_PER_W, row_body, 0)


def kernel(abilities, item_ids, disc_w, diff_w):
    ids32 = item_ids.astype(jnp.int32)
    diff_flat = diff_w.reshape(-1)

    mesh = plsc.VectorSubcoreMesh(core_axis_name="c", subcore_axis_name="s")

    run = pl.kernel(
        _mirt_body,
        out_type=jax.ShapeDtypeStruct((BATCH, HIST), jnp.float32),
        mesh=mesh,
        scratch_types=[
            pltpu.VMEM((HIST,), jnp.int32),
            pltpu.VMEM((HIST, NDIM), jnp.float32),
            pltpu.VMEM((HIST,), jnp.float32),
            pltpu.VMEM((HIST,), jnp.float32),
            pltpu.VMEM((ROWS_PER_W, NDIM), jnp.float32),
            pltpu.SemaphoreType.DMA,
            pltpu.SemaphoreType.DMA,
        ],
        name="neural_mirt_sc",
    )
    return run(abilities, ids32, disc_w, diff_flat)


# speedup vs baseline: 1.1277x; 1.1277x over previous
import jax
import jax.numpy as jnp
from jax import lax
from jax.experimental import pallas as pl
from jax.experimental.pallas import tpu as pltpu
from jax.experimental.pallas import tpu_sc as plsc

BATCH = 4096
HIST = 200
NDIM = 16
LANES = 16
NWORKERS = 32
ROWS_PER_W = BATCH // NWORKERS

_NCHUNKS = HIST // LANES + 1


def _mirt_body(abil_hbm, ids_hbm, disc_hbm, diff_hbm, out_hbm,
               idx_v, rows_v, diff_v, out_v, abil_v,
               sem_rows, sem_diff):
    nc = lax.axis_size("c")
    wid = lax.axis_index("s") * nc + lax.axis_index("c")
    base = wid * ROWS_PER_W

    pltpu.sync_copy(abil_hbm.at[pl.ds(base, ROWS_PER_W)], abil_v)

    def row_body(r, carry):
        row = base + r
        pltpu.sync_copy(ids_hbm.at[row], idx_v)
        cp_rows = pltpu.make_async_copy(disc_hbm.at[idx_v], rows_v, sem_rows)
        cp_diff = pltpu.make_async_copy(diff_hbm.at[idx_v], diff_v, sem_diff)
        cp_rows.start()
        cp_diff.start()
        cp_rows.wait()
        cp_diff.wait()

        abil = abil_v[r]

        def chunk_body(c, carry2):
            off = jnp.minimum(c * LANES, HIST - LANES)
            acc = jnp.zeros((LANES,), jnp.float32)
            lane = lax.iota(jnp.int32, LANES)
            for j in range(LANES):
                w = rows_v[off + j]
                s = plsc.cumsum(jnp.abs(w) * abil)[LANES - 1]
                acc = jnp.where(lane == j, s, acc)
            x = acc - diff_v[pl.ds(off, LANES)]
            out_v[pl.ds(off, LANES)] = 1.0 / (1.0 + jnp.exp(-x))
            return carry2

        lax.fori_loop(0, _NCHUNKS, chunk_body, 0)
        pltpu.sync_copy(out_v, out_hbm.at[row])
        return carry

    lax.fori_loop(0, ROWS_PER_W, row_body, 0)


def kernel(abilities, item_ids, disc_w, diff_w):
    ids32 = item_ids.astype(jnp.int32)
    diff_flat = diff_w.reshape(-1)

    mesh = plsc.VectorSubcoreMesh(core_axis_name="c", subcore_axis_name="s")

    run = pl.kernel(
        _mirt_body,
        out_type=jax.ShapeDtypeStruct((BATCH, HIST), jnp.float32),
        mesh=mesh,
        scratch_types=[
            pltpu.VMEM((HIST,), jnp.int32),
            pltpu.VMEM((HIST, NDIM), jnp.float32),
            pltpu.VMEM((HIST,), jnp.float32),
            pltpu.VMEM((HIST,), jnp.float32),
            pltpu.VMEM((ROWS_PER_W, NDIM), jnp.float32),
            pltpu.SemaphoreType.DMA,
            pltpu.SemaphoreType.DMA,
        ],
        compiler_params=pltpu.CompilerParams(needs_layout_passes=False,
                                              use_tc_tiling_on_sc=False),
        name="neural_mirt_sc",
    )
    return run(abilities, ids32, disc_w, diff_flat)


# speedup vs baseline: 1.4417x; 1.2784x over previous
import jax
import jax.numpy as jnp
from jax import lax
from jax.experimental import pallas as pl
from jax.experimental.pallas import tpu as pltpu
from jax.experimental.pallas import tpu_sc as plsc

BATCH = 4096
HIST = 200
NDIM = 16
LANES = 16
NWORKERS = 32
ROWS_PER_W = BATCH // NWORKERS
BLK_ROWS = 4
BLK = BLK_ROWS * HIST
NBLK = ROWS_PER_W // BLK_ROWS

_NCHUNKS = HIST // LANES + 1


def _start_block(i, ids_hbm, disc_hbm, diff_hbm, idx_v, rows_v, diff_v,
                 wbase, p, sem_rows, sem_diff):
    base = wbase + i * BLK
    pltpu.sync_copy(ids_hbm.at[pl.ds(base, BLK)], idx_v.at[p])
    pltpu.make_async_copy(disc_hbm.at[idx_v.at[p]], rows_v.at[p],
                          sem_rows).start()
    pltpu.make_async_copy(diff_hbm.at[idx_v.at[p]], diff_v.at[p],
                          sem_diff).start()


def _mirt_body(abil_hbm, ids_hbm, disc_hbm, diff_hbm, out_hbm,
               idx_v, rows_v, diff_v, out_v, abil_v,
               sem_rows, sem_diff):
    nc = lax.axis_size("c")
    wid = lax.axis_index("s") * nc + lax.axis_index("c")
    wbase = wid * ROWS_PER_W * HIST

    pltpu.sync_copy(abil_hbm.at[pl.ds(wid * ROWS_PER_W, ROWS_PER_W)], abil_v)

    _start_block(0, ids_hbm, disc_hbm, diff_hbm, idx_v, rows_v, diff_v,
                 wbase, 0, sem_rows, sem_diff)

    def block_body(i, carry):
        p = lax.rem(i, 2)
        pltpu.make_async_copy(disc_hbm.at[idx_v.at[p]], rows_v.at[p],
                              sem_rows).wait()
        pltpu.make_async_copy(diff_hbm.at[idx_v.at[p]], diff_v.at[p],
                              sem_diff).wait()

        @pl.when(i + 1 < NBLK)
        def _():
            _start_block(i + 1, ids_hbm, disc_hbm, diff_hbm,
                         idx_v, rows_v, diff_v, wbase, 1 - p,
                         sem_rows, sem_diff)

        lane = lax.iota(jnp.int32, LANES)

        def row_body(r, carry2):
            abil = abil_v[i * BLK_ROWS + r]
            rbase = r * HIST

            def chunk_body(c, carry3):
                off = rbase + jnp.minimum(c * LANES, HIST - LANES)
                acc = jnp.zeros((LANES,), jnp.float32)
                for j in range(LANES):
                    w = rows_v[p, off + j]
                    s = plsc.cumsum(jnp.abs(w) * abil)[LANES - 1]
                    acc = jnp.where(lane == j, s, acc)
                x = acc - diff_v[p, pl.ds(off, LANES)]
                out_v[p, pl.ds(off, LANES)] = 1.0 / (1.0 + jnp.exp(-x))
                return carry3

            return lax.fori_loop(0, _NCHUNKS, chunk_body, carry2)

        lax.fori_loop(0, BLK_ROWS, row_body, 0)
        pltpu.sync_copy(out_v.at[p], out_hbm.at[pl.ds(wbase + i * BLK, BLK)])
        return carry

    lax.fori_loop(0, NBLK, block_body, 0)


def kernel(abilities, item_ids, disc_w, diff_w):
    ids_flat = item_ids.astype(jnp.int32).reshape(-1)
    diff_flat = diff_w.reshape(-1)

    mesh = plsc.VectorSubcoreMesh(core_axis_name="c", subcore_axis_name="s")

    run = pl.kernel(
        _mirt_body,
        out_type=jax.ShapeDtypeStruct((BATCH * HIST,), jnp.float32),
        mesh=mesh,
        scratch_types=[
            pltpu.VMEM((2, BLK), jnp.int32),
            pltpu.VMEM((2, BLK, NDIM), jnp.float32),
            pltpu.VMEM((2, BLK), jnp.float32),
            pltpu.VMEM((2, BLK), jnp.float32),
            pltpu.VMEM((ROWS_PER_W, NDIM), jnp.float32),
            pltpu.SemaphoreType.DMA,
            pltpu.SemaphoreType.DMA,
        ],
        compiler_params=pltpu.CompilerParams(needs_layout_passes=False,
                                             use_tc_tiling_on_sc=False),
        name="neural_mirt_sc",
    )
    out_flat = run(abilities, ids_flat, disc_w, diff_flat)
    return out_flat.reshape(BATCH, HIST)
